# baseline (speedup 1.0000x reference)
import jax
import jax.numpy as jnp
from jax.experimental import pallas as pl
from jax.experimental.pallas import tpu as pltpu

_NCHUNKS = 8


def _dma_copy(w_ref, o_ref, sem):
    blk = o_ref.shape[0] // _NCHUNKS
    for i in range(_NCHUNKS):
        pltpu.async_copy(w_ref.at[pl.ds(i * blk, blk)],
                         o_ref.at[pl.ds(i * blk, blk)], sem)
    for i in range(_NCHUNKS):
        pltpu.make_async_copy(w_ref.at[pl.ds(i * blk, blk)],
                              o_ref.at[pl.ds(i * blk, blk)], sem).wait()


def kernel(input_ids, weights):
    seq_len = input_ids.shape[-1]
    dim = weights.shape[1]
    assert seq_len % _NCHUNKS == 0
    return pl.pallas_call(
        _dma_copy,
        in_specs=[pl.BlockSpec(memory_space=pltpu.MemorySpace.HBM)],
        out_specs=pl.BlockSpec(memory_space=pltpu.MemorySpace.HBM),
        out_shape=jax.ShapeDtypeStruct((seq_len, dim), weights.dtype),
        scratch_shapes=[pltpu.SemaphoreType.DMA],
    )(weights)


# speedup vs baseline: 49.0498x; 49.0498x over previous
import jax
import jax.numpy as jnp
from jax.experimental import pallas as pl


def _copy_block(w_ref, o_ref):
    o_ref[...] = w_ref[...]


def kernel(input_ids, weights):
    seq_len = input_ids.shape[-1]
    dim = weights.shape[1]
    blk = 4096
    assert seq_len % blk == 0
    return pl.pallas_call(
        _copy_block,
        grid=(seq_len // blk,),
        in_specs=[pl.BlockSpec((blk, dim), lambda i: (i, 0))],
        out_specs=pl.BlockSpec((blk, dim), lambda i: (i, 0)),
        out_shape=jax.ShapeDtypeStruct((seq_len, dim), weights.dtype),
    )(weights)
